# Initial kernel scaffold; baseline (speedup 1.0000x reference)
import jax
import jax.numpy as jnp
from jax.experimental import pallas as pl


def kernel(src_x, tar_x, params, src_edge, tar_edge):
    raise NotImplementedError("write your pallas kernel here")


# speedup vs baseline: 1.0128x; 1.0128x over previous
import jax
import jax.numpy as jnp
from jax.experimental import pallas as pl

_BN = 2000


def _mm_kernel(x_ref, w_ref, b_ref, o_ref):
    o_ref[...] = jax.lax.dot_general(
        x_ref[...], w_ref[...], (((1,), (0,)), ((), ())),
        preferred_element_type=jnp.float32,
        precision=jax.lax.Precision.DEFAULT) + b_ref[...]


def _mm(x, w, b):
    n, k = x.shape
    m = w.shape[1]
    assert n % _BN == 0
    return pl.pallas_call(
        _mm_kernel,
        grid=(n // _BN,),
        in_specs=[pl.BlockSpec((_BN, k), lambda i: (i, 0)),
                  pl.BlockSpec((k, m), lambda i: (0, 0)),
                  pl.BlockSpec((1, m), lambda i: (0, 0))],
        out_specs=pl.BlockSpec((_BN, m), lambda i: (i, 0)),
        out_shape=jax.ShapeDtypeStruct((n, m), jnp.float32),
    )(x, w, b.reshape(1, m))


def _proj(x, p):
    d = p['Wq'].shape[0]
    wcat = jnp.concatenate([p['Wq'].T, p['Wk'].T, p['Wv'].T, p['Ws'].T], axis=1)
    bcat = jnp.concatenate([p['bq'], p['bk'], p['bv'], p['bs']])
    out = _mm(x, wcat, bcat)
    return out[:, :d], out[:, d:2 * d], out[:, 2 * d:3 * d], out[:, 3 * d:]


def _tconv(x, edge_index, p):
    src, dst = edge_index[0], edge_index[1]
    n = x.shape[0]
    d = p['Wq'].shape[0]
    q, k, v, s = _proj(x, p)
    score = jnp.sum(q[dst] * k[src], axis=-1) / jnp.sqrt(jnp.float32(d))
    m = jax.ops.segment_max(score, dst, num_segments=n)
    m = jnp.where(jnp.isfinite(m), m, 0.0)
    e = jnp.exp(score - m[dst])
    denom = jax.ops.segment_sum(e, dst, num_segments=n)
    alpha = e / (denom[dst] + 1e-16)
    agg = jax.ops.segment_sum(alpha[:, None] * v[src], dst, num_segments=n)
    return agg + s


def _base_cls(x, edge_index, plist):
    h = x
    for p in plist:
        h = jax.nn.relu(_tconv(h, edge_index, p))
    return h


def _feat_ext(x, edge_index, fe):
    h1 = jax.nn.elu(_tconv(x, edge_index, fe[0]))
    h2 = jax.nn.elu(_tconv(h1, edge_index, fe[1]))
    return h2


def kernel(src_x, tar_x, params, src_edge, tar_edge):
    src_emb = _feat_ext(src_x, src_edge, params['fe'])
    tar_emb = _feat_ext(tar_x, tar_edge, params['fe'])
    src_pred = _base_cls(src_emb, src_edge, params['cls'])
    tar_pred = _base_cls(tar_emb, tar_edge, params['cls'])
    p_src = jax.nn.softmax(src_pred, axis=1)
    p_tar = jax.nn.softmax(tar_pred, axis=1)
    src_dom = _base_cls(src_emb, src_edge, params['gdc'])
    tar_dom = _base_cls(tar_emb, tar_edge, params['gdc'])
    s_out = []
    t_out = []
    for i in range(2):
        fs = p_src[:, i:i + 1] * src_emb
        s_out.append(_base_cls(fs, src_edge, params['dcis'][i]))
        ft = p_tar[:, i:i + 1] * tar_emb
        t_out.append(_base_cls(ft, tar_edge, params['dcis'][i]))
    return (src_pred, src_dom, tar_dom, s_out[0], s_out[1], t_out[0], t_out[1],
            src_emb, tar_emb)


# speedup vs baseline: 5.9804x; 5.9049x over previous
import functools
import math

import jax
import jax.numpy as jnp
from jax import lax
from jax.experimental import pallas as pl
from jax.experimental.pallas import tpu as pltpu
from jax.experimental.pallas import tpu_sc as plsc

N = 10000
NP = 10240
E = 320000
NC = 2
NS = 16
NW = NC * NS
EP = E // NW
C = 80
L = 16
ROWS = NP // NS

_BN = 2048


def _mm_kernel(x_ref, w_ref, b_ref, o_ref):
    o_ref[...] = jax.lax.dot_general(
        x_ref[...], w_ref[...], (((1,), (0,)), ((), ())),
        preferred_element_type=jnp.float32,
        precision=jax.lax.Precision.DEFAULT) + b_ref[...]


def _mm(x, w, b):
    n, k = x.shape
    m = w.shape[1]
    return pl.pallas_call(
        _mm_kernel,
        grid=(n // _BN,),
        in_specs=[pl.BlockSpec((_BN, k), lambda i: (i, 0)),
                  pl.BlockSpec((k, m), lambda i: (0, 0)),
                  pl.BlockSpec((1, m), lambda i: (0, 0))],
        out_specs=pl.BlockSpec((_BN, m), lambda i: (i, 0)),
        out_shape=jax.ShapeDtypeStruct((n, m), jnp.float32),
    )(x, w, b.reshape(1, m))


def _proj(x, p, dp):
    d = p['Wq'].shape[0]
    def padw(w):
        wt = w.T
        if dp != d:
            wt = jnp.pad(wt, ((0, 0), (0, dp - d)))
        return wt
    def padb(bb):
        if dp != d:
            bb = jnp.pad(bb, (0, dp - d))
        return bb
    wcat = jnp.concatenate([padw(p['Wq']), padw(p['Wk']),
                            padw(p['Wv']), padw(p['Ws'])], axis=1)
    bcat = jnp.concatenate([padb(p['bq']), padb(p['bk']),
                            padb(p['bv']), padb(p['bs'])])
    out = _mm(x, wcat, bcat)
    return (out[:, :dp], out[:, dp:2 * dp], out[:, 2 * dp:3 * dp],
            out[:, 3 * dp:])


def _lane_sum(x):
    lanes = lax.iota(jnp.int32, L)
    for sh in (8, 4, 2, 1):
        idx = jnp.bitwise_xor(lanes, sh)
        dnums = lax.GatherDimensionNumbers(
            offset_dims=(), collapsed_slice_dims=(0,), start_index_map=(0,))
        x = x + lax.gather(x, idx[:, None], dnums, (1,),
                           mode=lax.GatherScatterMode.PROMISE_IN_BOUNDS)
    return x


def _zero_rows(ref, nrows, ncols):
    z = jnp.zeros((L,), jnp.float32)
    def body(r, _):
        for t in range(ncols // L):
            ref[r, pl.ds(t * L, L)] = z
        return 0
    lax.fori_loop(0, nrows, body, 0)


def _sc_fused_body(dp, q_hbm, k_hbm, v_hbm, inv_hbm, src_hbm, dst_hbm,
                   agg_out, den_out, src_i, dst_i, q_rows, k_rows, v_rows,
                   w_buf, inv_v, agg_stage, den_stage, agg_sp, den_sp, sem):
    c = lax.axis_index("c")
    s = lax.axis_index("s")
    base = (c * NS + s) * EP

    pltpu.sync_copy(inv_hbm, inv_v)
    _zero_rows(agg_stage, ROWS, dp)
    _zero_rows(den_stage, ROWS, L)
    pltpu.sync_copy(agg_stage, agg_sp.at[pl.ds(s * ROWS, ROWS)])
    pltpu.sync_copy(den_stage, den_sp.at[pl.ds(s * ROWS, ROWS)])
    plsc.subcore_barrier()

    def chunk(i, _):
        off = base + i * C
        pltpu.sync_copy(src_hbm.at[pl.ds(off, C)], src_i)
        pltpu.sync_copy(dst_hbm.at[pl.ds(off, C)], dst_i)
        pltpu.async_copy(q_hbm.at[dst_i], q_rows, sem).wait()
        pltpu.async_copy(k_hbm.at[src_i], k_rows, sem).wait()
        pltpu.async_copy(v_hbm.at[src_i], v_rows, sem).wait()
        inv = inv_v[...]

        def edge(e, _):
            acc = q_rows[e, pl.ds(0, L)] * k_rows[e, pl.ds(0, L)]
            for t in range(1, dp // L):
                acc = acc + (q_rows[e, pl.ds(t * L, L)] *
                             k_rows[e, pl.ds(t * L, L)])
            w = jnp.exp(_lane_sum(acc) * inv)
            w_buf[e, :] = w
            for t in range(dp // L):
                v_rows[e, pl.ds(t * L, L)] = v_rows[e, pl.ds(t * L, L)] * w
            return 0
        lax.fori_loop(0, C, edge, 0)

        pltpu.sync_copy(v_rows, agg_sp.at[dst_i], add=True)
        pltpu.sync_copy(w_buf, den_sp.at[dst_i], add=True)
        return 0
    lax.fori_loop(0, EP // C, chunk, 0)

    plsc.subcore_barrier()
    pltpu.sync_copy(agg_sp.at[pl.ds(s * ROWS, ROWS)], agg_stage)
    pltpu.sync_copy(agg_stage, agg_out.at[c, pl.ds(s * ROWS, ROWS)])
    pltpu.sync_copy(den_sp.at[pl.ds(s * ROWS, ROWS)], den_stage)
    pltpu.sync_copy(den_stage, den_out.at[c, pl.ds(s * ROWS, ROWS)])


@functools.cache
def _sc_fused(dp):
    mesh = plsc.VectorSubcoreMesh(core_axis_name="c", subcore_axis_name="s")
    return pl.kernel(
        functools.partial(_sc_fused_body, dp),
        out_type=(jax.ShapeDtypeStruct((NC, NP, dp), jnp.float32),
                  jax.ShapeDtypeStruct((NC, NP, L), jnp.float32)),
        mesh=mesh,
        scratch_types=[
            pltpu.VMEM((C,), jnp.int32), pltpu.VMEM((C,), jnp.int32),
            pltpu.VMEM((C, dp), jnp.float32), pltpu.VMEM((C, dp), jnp.float32),
            pltpu.VMEM((C, dp), jnp.float32), pltpu.VMEM((C, L), jnp.float32),
            pltpu.VMEM((L,), jnp.float32),
            pltpu.VMEM((ROWS, dp), jnp.float32),
            pltpu.VMEM((ROWS, L), jnp.float32),
            pltpu.VMEM_SHARED((NP, dp), jnp.float32),
            pltpu.VMEM_SHARED((NP, L), jnp.float32),
            pltpu.SemaphoreType.DMA,
        ],
        compiler_params=pltpu.CompilerParams(use_tc_tiling_on_sc=False),
    )


def _sc_wden_body(dp, d_true, q_hbm, k_hbm, src_hbm, dst_hbm,
                  w_out, den_out, src_i, dst_i, q_rows, k_rows,
                  w_buf, den_stage, den_sp, sem):
    c = lax.axis_index("c")
    s = lax.axis_index("s")
    base = (c * NS + s) * EP
    inv = 1.0 / math.sqrt(float(d_true))

    _zero_rows(den_stage, ROWS, L)
    pltpu.sync_copy(den_stage, den_sp.at[pl.ds(s * ROWS, ROWS)])
    plsc.subcore_barrier()

    def chunk(i, _):
        off = base + i * C
        pltpu.sync_copy(src_hbm.at[pl.ds(off, C)], src_i)
        pltpu.sync_copy(dst_hbm.at[pl.ds(off, C)], dst_i)
        pltpu.async_copy(q_hbm.at[dst_i], q_rows, sem).wait()
        pltpu.async_copy(k_hbm.at[src_i], k_rows, sem).wait()

        def edge(e, _):
            acc = q_rows[e, pl.ds(0, L)] * k_rows[e, pl.ds(0, L)]
            for t in range(1, dp // L):
                acc = acc + (q_rows[e, pl.ds(t * L, L)] *
                             k_rows[e, pl.ds(t * L, L)])
            w_buf[e, :] = jnp.exp(_lane_sum(acc) * inv)
            return 0
        lax.fori_loop(0, C, edge, 0)

        pltpu.sync_copy(w_buf, w_out.at[pl.ds(off, C)])
        pltpu.sync_copy(w_buf, den_sp.at[dst_i], add=True)
        return 0
    lax.fori_loop(0, EP // C, chunk, 0)

    plsc.subcore_barrier()
    pltpu.sync_copy(den_sp.at[pl.ds(s * ROWS, ROWS)], den_stage)
    pltpu.sync_copy(den_stage, den_out.at[c, pl.ds(s * ROWS, ROWS)])


@functools.cache
def _sc_wden(dp, d_true):
    mesh = plsc.VectorSubcoreMesh(core_axis_name="c", subcore_axis_name="s")
    return pl.kernel(
        functools.partial(_sc_wden_body, dp, d_true),
        out_type=(jax.ShapeDtypeStruct((E, L), jnp.float32),
                  jax.ShapeDtypeStruct((NC, NP, L), jnp.float32)),
        mesh=mesh,
        scratch_types=[
            pltpu.VMEM((C,), jnp.int32), pltpu.VMEM((C,), jnp.int32),
            pltpu.VMEM((C, dp), jnp.float32), pltpu.VMEM((C, dp), jnp.float32),
            pltpu.VMEM((C, L), jnp.float32),
            pltpu.VMEM((ROWS, L), jnp.float32),
            pltpu.VMEM_SHARED((NP, L), jnp.float32),
            pltpu.SemaphoreType.DMA,
        ],
        compiler_params=pltpu.CompilerParams(use_tc_tiling_on_sc=False),
    )


def _sc_agg_body(dpc, ncc, v_hbm, src_hbm, dst_hbm, w_hbm,
                 agg_out, src_i, dst_i, idx2, v_rows, w_buf, agg_stage,
                 agg_sp, sem):
    c = lax.axis_index("c")
    s = lax.axis_index("s")
    base = (c * NS + s) * EP

    _zero_rows(agg_stage, ROWS, dpc)
    for j in range(ncc):
        pltpu.sync_copy(agg_stage, agg_sp.at[pl.ds(s * ROWS, ROWS)])
        plsc.subcore_barrier()

        def chunk(i, _):
            off = base + i * C
            pltpu.sync_copy(src_hbm.at[pl.ds(off, C)], src_i)
            pltpu.sync_copy(dst_hbm.at[pl.ds(off, C)], dst_i)
            pltpu.sync_copy(w_hbm.at[pl.ds(off, C)], w_buf)
            for t in range(C // L):
                idx2[pl.ds(t * L, L)] = src_i[pl.ds(t * L, L)] * ncc + j
            pltpu.async_copy(v_hbm.at[idx2], v_rows, sem).wait()

            def edge(e, _):
                w = w_buf[e, :]
                for t in range(dpc // L):
                    v_rows[e, pl.ds(t * L, L)] = (
                        v_rows[e, pl.ds(t * L, L)] * w)
                return 0
            lax.fori_loop(0, C, edge, 0)

            pltpu.sync_copy(v_rows, agg_sp.at[dst_i], add=True)
            return 0
        lax.fori_loop(0, EP // C, chunk, 0)

        plsc.subcore_barrier()
        pltpu.sync_copy(agg_sp.at[pl.ds(s * ROWS, ROWS)], agg_stage)
        pltpu.sync_copy(
            agg_stage, agg_out.at[c, pl.ds(j * NP + s * ROWS, ROWS)])
        if j != ncc - 1:
            _zero_rows(agg_stage, ROWS, dpc)


@functools.cache
def _sc_agg(dpc, ncc):
    mesh = plsc.VectorSubcoreMesh(core_axis_name="c", subcore_axis_name="s")
    return pl.kernel(
        functools.partial(_sc_agg_body, dpc, ncc),
        out_type=jax.ShapeDtypeStruct((NC, ncc * NP, dpc), jnp.float32),
        mesh=mesh,
        scratch_types=[
            pltpu.VMEM((C,), jnp.int32), pltpu.VMEM((C,), jnp.int32),
            pltpu.VMEM((C,), jnp.int32),
            pltpu.VMEM((C, dpc), jnp.float32),
            pltpu.VMEM((C, L), jnp.float32),
            pltpu.VMEM((ROWS, dpc), jnp.float32),
            pltpu.VMEM_SHARED((NP, dpc), jnp.float32),
            pltpu.SemaphoreType.DMA,
        ],
        compiler_params=pltpu.CompilerParams(use_tc_tiling_on_sc=False),
    )


def _combine_kernel(act, a_ref, d_ref, s_ref, o_ref):
    den = d_ref[0, :, 0:1] + d_ref[1, :, 0:1] + 1e-16
    val = (a_ref[0, 0] + a_ref[1, 0]) / den + s_ref[0]
    if act == 'relu':
        val = jnp.maximum(val, 0.0)
    elif act == 'elu':
        val = jnp.where(val > 0, val, jnp.exp(val) - 1.0)
    o_ref[0] = val


def _combine(agg, den, s_arr, act, ncc, dpc):
    dp = ncc * dpc
    agg = agg.reshape(NC, ncc, NP, dpc)
    s_t = s_arr.reshape(NP, ncc, dpc).transpose(1, 0, 2)
    out = pl.pallas_call(
        functools.partial(_combine_kernel, act),
        grid=(ncc, NP // _BN),
        in_specs=[
            pl.BlockSpec((NC, 1, _BN, dpc), lambda j, i: (0, j, i, 0)),
            pl.BlockSpec((NC, _BN, L), lambda j, i: (0, i, 0)),
            pl.BlockSpec((1, _BN, dpc), lambda j, i: (j, i, 0)),
        ],
        out_specs=pl.BlockSpec((1, _BN, dpc), lambda j, i: (j, i, 0)),
        out_shape=jax.ShapeDtypeStruct((ncc, NP, dpc), jnp.float32),
    )(agg, den, s_t)
    return out.transpose(1, 0, 2).reshape(NP, dp)


def _smscale_kernel(p_ref, e_ref, o0_ref, o1_ref):
    x0 = p_ref[:, 0:1]
    x1 = p_ref[:, 1:2]
    m = jnp.maximum(x0, x1)
    e0 = jnp.exp(x0 - m)
    e1 = jnp.exp(x1 - m)
    p0 = e0 / (e0 + e1)
    emb = e_ref[...]
    o0_ref[...] = p0 * emb
    o1_ref[...] = (1.0 - p0) * emb


def _smscale(pred, emb):
    nn, dd = emb.shape
    dw = pred.shape[1]
    return pl.pallas_call(
        _smscale_kernel,
        grid=(nn // _BN,),
        in_specs=[pl.BlockSpec((_BN, dw), lambda i: (i, 0)),
                  pl.BlockSpec((_BN, dd), lambda i: (i, 0))],
        out_specs=[pl.BlockSpec((_BN, dd), lambda i: (i, 0)),
                   pl.BlockSpec((_BN, dd), lambda i: (i, 0))],
        out_shape=[jax.ShapeDtypeStruct((nn, dd), jnp.float32),
                   jax.ShapeDtypeStruct((nn, dd), jnp.float32)],
    )(pred, emb)


def _tconv(x, esrc, edst, p, act):
    d = p['Wq'].shape[0]
    dp = 32
    q, k, v, s = _proj(x, p, dp)
    inv = jnp.full((L,), 1.0 / math.sqrt(float(d)), jnp.float32)
    agg, den = _sc_fused(dp)(q, k, v, inv, esrc, edst)
    return _combine(agg, den, s, act, 1, dp)


def _tconv64(x, esrc, edst, p, act):
    d = p['Wq'].shape[0]
    q, k, v, s = _proj(x, p, d)
    inv = jnp.full((L,), 1.0 / math.sqrt(float(d)), jnp.float32)
    agg, den = _sc_fused(64)(q, k, v, inv, esrc, edst)
    return _combine(agg, den, s, act, 1, d)


def _tconv_wide(x, esrc, edst, p, act):
    d = p['Wq'].shape[0]
    dpc = 32
    ncc = d // dpc
    q, k, v, s = _proj(x, p, d)
    w_e, den = _sc_wden(d, d)(q, k, esrc, edst)
    v_view = v.reshape(NP * ncc, dpc)
    agg = _sc_agg(dpc, ncc)(v_view, esrc, edst, w_e)
    return _combine(agg, den, s, act, ncc, dpc)


def _head(x, esrc, edst, plist):
    h = x
    for p in plist:
        h = _tconv(h, esrc, edst, p, 'relu')
    return h


def _graph_forward(x, edge, params):
    esrc = edge[0].astype(jnp.int32)
    edst = edge[1].astype(jnp.int32)
    h = _tconv_wide(x, esrc, edst, params['fe'][0], 'elu')
    emb = _tconv64(h, esrc, edst, params['fe'][1], 'elu')
    pred = _head(emb, esrc, edst, params['cls'])
    dom = _head(emb, esrc, edst, params['gdc'])
    fs0, fs1 = _smscale(pred, emb)
    d0 = _head(fs0, esrc, edst, params['dcis'][0])
    d1 = _head(fs1, esrc, edst, params['dcis'][1])
    return pred, dom, d0, d1, emb


def kernel(src_x, tar_x, params, src_edge, tar_edge):
    xs = jnp.pad(src_x, ((0, NP - N), (0, 0)))
    xt = jnp.pad(tar_x, ((0, NP - N), (0, 0)))

    s_pred, s_dom, s_d0, s_d1, s_emb = _graph_forward(xs, src_edge, params)
    t_pred, t_dom, t_d0, t_d1, t_emb = _graph_forward(xt, tar_edge, params)

    return (s_pred[:N, :2], s_dom[:N, :2], t_dom[:N, :2],
            s_d0[:N, :2], s_d1[:N, :2], t_d0[:N, :2], t_d1[:N, :2],
            s_emb[:N], t_emb[:N])


# speedup vs baseline: 6.8416x; 1.1440x over previous
import functools
import math

import jax
import jax.numpy as jnp
from jax import lax
from jax.experimental import pallas as pl
from jax.experimental.pallas import tpu as pltpu
from jax.experimental.pallas import tpu_sc as plsc

N = 10000
NP = 10240
E = 320000
NC = 2
NS = 16
NW = NC * NS
EP = E // NW
C = 80
L = 16
ROWS = NP // NS

_BN = 2048


def _mm_kernel(x_ref, w_ref, b_ref, o_ref):
    o_ref[...] = jax.lax.dot_general(
        x_ref[...], w_ref[...], (((1,), (0,)), ((), ())),
        preferred_element_type=jnp.float32,
        precision=jax.lax.Precision.DEFAULT) + b_ref[...]


def _mm(x, w, b):
    n, k = x.shape
    m = w.shape[1]
    return pl.pallas_call(
        _mm_kernel,
        grid=(n // _BN,),
        in_specs=[pl.BlockSpec((_BN, k), lambda i: (i, 0)),
                  pl.BlockSpec((k, m), lambda i: (0, 0)),
                  pl.BlockSpec((1, m), lambda i: (0, 0))],
        out_specs=pl.BlockSpec((_BN, m), lambda i: (i, 0)),
        out_shape=jax.ShapeDtypeStruct((n, m), jnp.float32),
    )(x, w, b.reshape(1, m))


def _proj(x, p, dp):
    d = p['Wq'].shape[0]
    def padw(w):
        wt = w.T
        if dp != d:
            wt = jnp.pad(wt, ((0, 0), (0, dp - d)))
        return wt
    def padb(bb):
        if dp != d:
            bb = jnp.pad(bb, (0, dp - d))
        return bb
    wcat = jnp.concatenate([padw(p['Wq']), padw(p['Wk']),
                            padw(p['Wv']), padw(p['Ws'])], axis=1)
    bcat = jnp.concatenate([padb(p['bq']), padb(p['bk']),
                            padb(p['bv']), padb(p['bs'])])
    out = _mm(x, wcat, bcat)
    return (out[:, :dp], out[:, dp:2 * dp], out[:, 2 * dp:3 * dp],
            out[:, 3 * dp:])


def _lane_sum(x):
    lanes = lax.iota(jnp.int32, L)
    for sh in (8, 4, 2, 1):
        idx = jnp.bitwise_xor(lanes, sh)
        dnums = lax.GatherDimensionNumbers(
            offset_dims=(), collapsed_slice_dims=(0,), start_index_map=(0,))
        x = x + lax.gather(x, idx[:, None], dnums, (1,),
                           mode=lax.GatherScatterMode.PROMISE_IN_BOUNDS)
    return x


def _zero_rows(ref, nrows, ncols):
    z = jnp.zeros((L,), jnp.float32)
    def body(r, _):
        for t in range(ncols // L):
            ref[r, pl.ds(t * L, L)] = z
        return 0
    lax.fori_loop(0, nrows, body, 0)


def _sc_fused_body(dp, q_hbm, k_hbm, v_hbm, inv_hbm, src_hbm, dst_hbm,
                   agg_out, den_out, src_i, dst_i, q_rows, k_rows, v_rows,
                   w_buf, inv_v, agg_stage, den_stage, agg_sp, den_sp, sem):
    c = lax.axis_index("c")
    s = lax.axis_index("s")
    base = (c * NS + s) * EP

    pltpu.sync_copy(inv_hbm, inv_v)
    _zero_rows(agg_stage, ROWS, dp)
    _zero_rows(den_stage, ROWS, L)
    pltpu.sync_copy(agg_stage, agg_sp.at[pl.ds(s * ROWS, ROWS)])
    pltpu.sync_copy(den_stage, den_sp.at[pl.ds(s * ROWS, ROWS)])
    plsc.subcore_barrier()

    def chunk(i, _):
        off = base + i * C
        pltpu.sync_copy(src_hbm.at[pl.ds(off, C)], src_i)
        pltpu.sync_copy(dst_hbm.at[pl.ds(off, C)], dst_i)
        dq = pltpu.async_copy(q_hbm.at[dst_i], q_rows, sem)
        dk = pltpu.async_copy(k_hbm.at[src_i], k_rows, sem)
        dv = pltpu.async_copy(v_hbm.at[src_i], v_rows, sem)
        dq.wait()
        dk.wait()
        dv.wait()
        inv = inv_v[...]

        def edge(e, _):
            acc = q_rows[e, pl.ds(0, L)] * k_rows[e, pl.ds(0, L)]
            for t in range(1, dp // L):
                acc = acc + (q_rows[e, pl.ds(t * L, L)] *
                             k_rows[e, pl.ds(t * L, L)])
            w = jnp.exp(_lane_sum(acc) * inv)
            w_buf[e, :] = w
            for t in range(dp // L):
                v_rows[e, pl.ds(t * L, L)] = v_rows[e, pl.ds(t * L, L)] * w
            return 0
        lax.fori_loop(0, C, edge, 0)

        pltpu.sync_copy(v_rows, agg_sp.at[dst_i], add=True)
        pltpu.sync_copy(w_buf, den_sp.at[dst_i], add=True)
        return 0
    lax.fori_loop(0, EP // C, chunk, 0)

    plsc.subcore_barrier()
    pltpu.sync_copy(agg_sp.at[pl.ds(s * ROWS, ROWS)], agg_stage)
    pltpu.sync_copy(agg_stage, agg_out.at[c, pl.ds(s * ROWS, ROWS)])
    pltpu.sync_copy(den_sp.at[pl.ds(s * ROWS, ROWS)], den_stage)
    pltpu.sync_copy(den_stage, den_out.at[c, pl.ds(s * ROWS, ROWS)])


@functools.cache
def _sc_fused(dp):
    mesh = plsc.VectorSubcoreMesh(core_axis_name="c", subcore_axis_name="s")
    return pl.kernel(
        functools.partial(_sc_fused_body, dp),
        out_type=(jax.ShapeDtypeStruct((NC, NP, dp), jnp.float32),
                  jax.ShapeDtypeStruct((NC, NP, L), jnp.float32)),
        mesh=mesh,
        scratch_types=[
            pltpu.VMEM((C,), jnp.int32), pltpu.VMEM((C,), jnp.int32),
            pltpu.VMEM((C, dp), jnp.float32), pltpu.VMEM((C, dp), jnp.float32),
            pltpu.VMEM((C, dp), jnp.float32), pltpu.VMEM((C, L), jnp.float32),
            pltpu.VMEM((L,), jnp.float32),
            pltpu.VMEM((ROWS, dp), jnp.float32),
            pltpu.VMEM((ROWS, L), jnp.float32),
            pltpu.VMEM_SHARED((NP, dp), jnp.float32),
            pltpu.VMEM_SHARED((NP, L), jnp.float32),
            pltpu.SemaphoreType.DMA,
        ],
        compiler_params=pltpu.CompilerParams(use_tc_tiling_on_sc=False),
    )


def _sc_wden_body(dp, d_true, q_hbm, k_hbm, src_hbm, dst_hbm,
                  w_out, den_out, src_i, dst_i, q_rows, k_rows,
                  w_buf, den_stage, den_sp, sem):
    c = lax.axis_index("c")
    s = lax.axis_index("s")
    base = (c * NS + s) * EP
    inv = 1.0 / math.sqrt(float(d_true))

    _zero_rows(den_stage, ROWS, L)
    pltpu.sync_copy(den_stage, den_sp.at[pl.ds(s * ROWS, ROWS)])
    plsc.subcore_barrier()

    def chunk(i, _):
        off = base + i * C
        pltpu.sync_copy(src_hbm.at[pl.ds(off, C)], src_i)
        pltpu.sync_copy(dst_hbm.at[pl.ds(off, C)], dst_i)
        dq = pltpu.async_copy(q_hbm.at[dst_i], q_rows, sem)
        dk = pltpu.async_copy(k_hbm.at[src_i], k_rows, sem)
        dq.wait()
        dk.wait()

        def edge(e, _):
            acc = q_rows[e, pl.ds(0, L)] * k_rows[e, pl.ds(0, L)]
            for t in range(1, dp // L):
                acc = acc + (q_rows[e, pl.ds(t * L, L)] *
                             k_rows[e, pl.ds(t * L, L)])
            w_buf[e, :] = jnp.exp(_lane_sum(acc) * inv)
            return 0
        lax.fori_loop(0, C, edge, 0)

        pltpu.sync_copy(w_buf, w_out.at[pl.ds(off, C)])
        pltpu.sync_copy(w_buf, den_sp.at[dst_i], add=True)
        return 0
    lax.fori_loop(0, EP // C, chunk, 0)

    plsc.subcore_barrier()
    pltpu.sync_copy(den_sp.at[pl.ds(s * ROWS, ROWS)], den_stage)
    pltpu.sync_copy(den_stage, den_out.at[c, pl.ds(s * ROWS, ROWS)])


@functools.cache
def _sc_wden(dp, d_true):
    mesh = plsc.VectorSubcoreMesh(core_axis_name="c", subcore_axis_name="s")
    return pl.kernel(
        functools.partial(_sc_wden_body, dp, d_true),
        out_type=(jax.ShapeDtypeStruct((E, L), jnp.float32),
                  jax.ShapeDtypeStruct((NC, NP, L), jnp.float32)),
        mesh=mesh,
        scratch_types=[
            pltpu.VMEM((C,), jnp.int32), pltpu.VMEM((C,), jnp.int32),
            pltpu.VMEM((C, dp), jnp.float32), pltpu.VMEM((C, dp), jnp.float32),
            pltpu.VMEM((C, L), jnp.float32),
            pltpu.VMEM((ROWS, L), jnp.float32),
            pltpu.VMEM_SHARED((NP, L), jnp.float32),
            pltpu.SemaphoreType.DMA,
        ],
        compiler_params=pltpu.CompilerParams(use_tc_tiling_on_sc=False),
    )


def _sc_agg_body(dpc, ncc, v_hbm, src_hbm, dst_hbm, w_hbm,
                 agg_out, src_i, dst_i, idx2, v_rows, w_buf, agg_stage,
                 agg_sp, sem):
    c = lax.axis_index("c")
    s = lax.axis_index("s")
    base = (c * NS + s) * EP

    _zero_rows(agg_stage, ROWS, dpc)
    for j in range(ncc):
        pltpu.sync_copy(agg_stage, agg_sp.at[pl.ds(s * ROWS, ROWS)])
        plsc.subcore_barrier()

        def chunk(i, _):
            off = base + i * C
            pltpu.sync_copy(src_hbm.at[pl.ds(off, C)], src_i)
            pltpu.sync_copy(dst_hbm.at[pl.ds(off, C)], dst_i)
            pltpu.sync_copy(w_hbm.at[pl.ds(off, C)], w_buf)
            for t in range(C // L):
                idx2[pl.ds(t * L, L)] = src_i[pl.ds(t * L, L)] * ncc + j
            pltpu.async_copy(v_hbm.at[idx2], v_rows, sem).wait()

            def edge(e, _):
                w = w_buf[e, :]
                for t in range(dpc // L):
                    v_rows[e, pl.ds(t * L, L)] = (
                        v_rows[e, pl.ds(t * L, L)] * w)
                return 0
            lax.fori_loop(0, C, edge, 0)

            pltpu.sync_copy(v_rows, agg_sp.at[dst_i], add=True)
            return 0
        lax.fori_loop(0, EP // C, chunk, 0)

        plsc.subcore_barrier()
        pltpu.sync_copy(agg_sp.at[pl.ds(s * ROWS, ROWS)], agg_stage)
        pltpu.sync_copy(
            agg_stage, agg_out.at[c, pl.ds(j * NP + s * ROWS, ROWS)])
        if j != ncc - 1:
            _zero_rows(agg_stage, ROWS, dpc)


@functools.cache
def _sc_agg(dpc, ncc):
    mesh = plsc.VectorSubcoreMesh(core_axis_name="c", subcore_axis_name="s")
    return pl.kernel(
        functools.partial(_sc_agg_body, dpc, ncc),
        out_type=jax.ShapeDtypeStruct((NC, ncc * NP, dpc), jnp.float32),
        mesh=mesh,
        scratch_types=[
            pltpu.VMEM((C,), jnp.int32), pltpu.VMEM((C,), jnp.int32),
            pltpu.VMEM((C,), jnp.int32),
            pltpu.VMEM((C, dpc), jnp.float32),
            pltpu.VMEM((C, L), jnp.float32),
            pltpu.VMEM((ROWS, dpc), jnp.float32),
            pltpu.VMEM_SHARED((NP, dpc), jnp.float32),
            pltpu.SemaphoreType.DMA,
        ],
        compiler_params=pltpu.CompilerParams(use_tc_tiling_on_sc=False),
    )


def _combine_kernel(act, a_ref, d_ref, s_ref, o_ref):
    den = d_ref[0, :, 0:1] + d_ref[1, :, 0:1] + 1e-16
    val = (a_ref[0, 0] + a_ref[1, 0]) / den + s_ref[0]
    if act == 'relu':
        val = jnp.maximum(val, 0.0)
    elif act == 'elu':
        val = jnp.where(val > 0, val, jnp.exp(val) - 1.0)
    o_ref[0] = val


def _combine(agg, den, s_arr, act, ncc, dpc):
    dp = ncc * dpc
    agg = agg.reshape(NC, ncc, NP, dpc)
    s_t = s_arr.reshape(NP, ncc, dpc).transpose(1, 0, 2)
    out = pl.pallas_call(
        functools.partial(_combine_kernel, act),
        grid=(ncc, NP // _BN),
        in_specs=[
            pl.BlockSpec((NC, 1, _BN, dpc), lambda j, i: (0, j, i, 0)),
            pl.BlockSpec((NC, _BN, L), lambda j, i: (0, i, 0)),
            pl.BlockSpec((1, _BN, dpc), lambda j, i: (j, i, 0)),
        ],
        out_specs=pl.BlockSpec((1, _BN, dpc), lambda j, i: (j, i, 0)),
        out_shape=jax.ShapeDtypeStruct((ncc, NP, dpc), jnp.float32),
    )(agg, den, s_t)
    return out.transpose(1, 0, 2).reshape(NP, dp)


def _smscale_kernel(p_ref, e_ref, o0_ref, o1_ref):
    x0 = p_ref[:, 0:1]
    x1 = p_ref[:, 1:2]
    m = jnp.maximum(x0, x1)
    e0 = jnp.exp(x0 - m)
    e1 = jnp.exp(x1 - m)
    p0 = e0 / (e0 + e1)
    emb = e_ref[...]
    o0_ref[...] = p0 * emb
    o1_ref[...] = (1.0 - p0) * emb


def _smscale(pred, emb):
    nn, dd = emb.shape
    dw = pred.shape[1]
    return pl.pallas_call(
        _smscale_kernel,
        grid=(nn // _BN,),
        in_specs=[pl.BlockSpec((_BN, dw), lambda i: (i, 0)),
                  pl.BlockSpec((_BN, dd), lambda i: (i, 0))],
        out_specs=[pl.BlockSpec((_BN, dd), lambda i: (i, 0)),
                   pl.BlockSpec((_BN, dd), lambda i: (i, 0))],
        out_shape=[jax.ShapeDtypeStruct((nn, dd), jnp.float32),
                   jax.ShapeDtypeStruct((nn, dd), jnp.float32)],
    )(pred, emb)


def _tconv(x, esrc, edst, p, act):
    d = p['Wq'].shape[0]
    dp = 32
    q, k, v, s = _proj(x, p, dp)
    inv = jnp.full((L,), 1.0 / math.sqrt(float(d)), jnp.float32)
    agg, den = _sc_fused(dp)(q, k, v, inv, esrc, edst)
    return _combine(agg, den, s, act, 1, dp)


def _tconv64(x, esrc, edst, p, act):
    d = p['Wq'].shape[0]
    q, k, v, s = _proj(x, p, d)
    inv = jnp.full((L,), 1.0 / math.sqrt(float(d)), jnp.float32)
    agg, den = _sc_fused(64)(q, k, v, inv, esrc, edst)
    return _combine(agg, den, s, act, 1, d)


def _tconv_wide(x, esrc, edst, p, act):
    d = p['Wq'].shape[0]
    dpc = 32
    ncc = d // dpc
    q, k, v, s = _proj(x, p, d)
    w_e, den = _sc_wden(d, d)(q, k, esrc, edst)
    v_view = v.reshape(NP * ncc, dpc)
    agg = _sc_agg(dpc, ncc)(v_view, esrc, edst, w_e)
    return _combine(agg, den, s, act, ncc, dpc)


def _head(x, esrc, edst, plist):
    h = x
    for p in plist:
        h = _tconv(h, esrc, edst, p, 'relu')
    return h


def _graph_forward(x, edge, params):
    esrc = edge[0].astype(jnp.int32)
    edst = edge[1].astype(jnp.int32)
    h = _tconv_wide(x, esrc, edst, params['fe'][0], 'elu')
    emb = _tconv64(h, esrc, edst, params['fe'][1], 'elu')
    pred = _head(emb, esrc, edst, params['cls'])
    dom = _head(emb, esrc, edst, params['gdc'])
    fs0, fs1 = _smscale(pred, emb)
    d0 = _head(fs0, esrc, edst, params['dcis'][0])
    d1 = _head(fs1, esrc, edst, params['dcis'][1])
    return pred, dom, d0, d1, emb


def kernel(src_x, tar_x, params, src_edge, tar_edge):
    xs = jnp.pad(src_x, ((0, NP - N), (0, 0)))
    xt = jnp.pad(tar_x, ((0, NP - N), (0, 0)))

    s_pred, s_dom, s_d0, s_d1, s_emb = _graph_forward(xs, src_edge, params)
    t_pred, t_dom, t_d0, t_d1, t_emb = _graph_forward(xt, tar_edge, params)

    return (s_pred[:N, :2], s_dom[:N, :2], t_dom[:N, :2],
            s_d0[:N, :2], s_d1[:N, :2], t_d0[:N, :2], t_d1[:N, :2],
            s_emb[:N], t_emb[:N])


# speedup vs baseline: 7.5293x; 1.1005x over previous
import functools
import math

import jax
import jax.numpy as jnp
from jax import lax
from jax.experimental import pallas as pl
from jax.experimental.pallas import tpu as pltpu
from jax.experimental.pallas import tpu_sc as plsc

N = 10000
NP = 10240
E = 320000
NC = 2
NS = 16
NW = NC * NS
EP = E // NW
C = 80
L = 16
ROWS = NP // NS

_BN = 2048


def _mm_kernel(x_ref, w_ref, b_ref, o_ref):
    o_ref[...] = jax.lax.dot_general(
        x_ref[...], w_ref[...], (((1,), (0,)), ((), ())),
        preferred_element_type=jnp.float32,
        precision=jax.lax.Precision.DEFAULT) + b_ref[...]


def _mm(x, w, b):
    n, k = x.shape
    m = w.shape[1]
    return pl.pallas_call(
        _mm_kernel,
        grid=(n // _BN,),
        in_specs=[pl.BlockSpec((_BN, k), lambda i: (i, 0)),
                  pl.BlockSpec((k, m), lambda i: (0, 0)),
                  pl.BlockSpec((1, m), lambda i: (0, 0))],
        out_specs=pl.BlockSpec((_BN, m), lambda i: (i, 0)),
        out_shape=jax.ShapeDtypeStruct((n, m), jnp.float32),
    )(x, w, b.reshape(1, m))


def _proj(x, p, dp):
    d = p['Wq'].shape[0]
    def padw(w):
        wt = w.T
        if dp != d:
            wt = jnp.pad(wt, ((0, 0), (0, dp - d)))
        return wt
    def padb(bb):
        if dp != d:
            bb = jnp.pad(bb, (0, dp - d))
        return bb
    wcat = jnp.concatenate([padw(p['Wq']), padw(p['Wk']),
                            padw(p['Wv']), padw(p['Ws'])], axis=1)
    bcat = jnp.concatenate([padb(p['bq']), padb(p['bk']),
                            padb(p['bv']), padb(p['bs'])])
    out = _mm(x, wcat, bcat)
    return (out[:, :dp], out[:, dp:2 * dp], out[:, 2 * dp:3 * dp],
            out[:, 3 * dp:])


def _lane_sum(x):
    lanes = lax.iota(jnp.int32, L)
    for sh in (8, 4, 2, 1):
        idx = jnp.bitwise_xor(lanes, sh)
        dnums = lax.GatherDimensionNumbers(
            offset_dims=(), collapsed_slice_dims=(0,), start_index_map=(0,))
        x = x + lax.gather(x, idx[:, None], dnums, (1,),
                           mode=lax.GatherScatterMode.PROMISE_IN_BOUNDS)
    return x


def _zero_rows(ref, nrows, ncols):
    z = jnp.zeros((L,), jnp.float32)
    def body(r, _):
        for t in range(ncols // L):
            ref[r, pl.ds(t * L, L)] = z
        return 0
    lax.fori_loop(0, nrows, body, 0)


def _sc_fused_body(dp, nbuf, q_hbm, k_hbm, v_hbm, inv_hbm, src_hbm, dst_hbm,
                   agg_out, den_out, *scr):
    (inv_v, agg_stage, den_stage, agg_sp, den_sp) = scr[:5]
    bufs = []
    for b in range(nbuf):
        bufs.append(scr[5 + 7 * b: 5 + 7 * (b + 1)])
    c = lax.axis_index("c")
    s = lax.axis_index("s")
    base = (c * NS + s) * EP

    pltpu.sync_copy(inv_hbm, inv_v)
    _zero_rows(agg_stage, ROWS, dp)
    _zero_rows(den_stage, ROWS, L)
    pltpu.sync_copy(agg_stage, agg_sp.at[pl.ds(s * ROWS, ROWS)])
    pltpu.sync_copy(den_stage, den_sp.at[pl.ds(s * ROWS, ROWS)])
    plsc.subcore_barrier()

    def issue(ci, b):
        src_i, dst_i, q_rows, k_rows, v_rows, w_buf, sem = bufs[b]
        off = base + ci * C
        pltpu.sync_copy(src_hbm.at[pl.ds(off, C)], src_i)
        pltpu.sync_copy(dst_hbm.at[pl.ds(off, C)], dst_i)
        return (pltpu.async_copy(q_hbm.at[dst_i], q_rows, sem),
                pltpu.async_copy(k_hbm.at[src_i], k_rows, sem),
                pltpu.async_copy(v_hbm.at[src_i], v_rows, sem))

    def finish(descs, b):
        src_i, dst_i, q_rows, k_rows, v_rows, w_buf, sem = bufs[b]
        for dsc in descs:
            dsc.wait()
        inv = inv_v[...]

        def edge(e, _):
            acc = q_rows[e, pl.ds(0, L)] * k_rows[e, pl.ds(0, L)]
            for t in range(1, dp // L):
                acc = acc + (q_rows[e, pl.ds(t * L, L)] *
                             k_rows[e, pl.ds(t * L, L)])
            w = jnp.exp(_lane_sum(acc) * inv)
            w_buf[e, :] = w
            for t in range(dp // L):
                v_rows[e, pl.ds(t * L, L)] = v_rows[e, pl.ds(t * L, L)] * w
            return 0
        lax.fori_loop(0, C, edge, 0)

        pltpu.sync_copy(v_rows, agg_sp.at[dst_i], add=True)
        pltpu.sync_copy(w_buf, den_sp.at[dst_i], add=True)

    nch = EP // C
    if nbuf == 1:
        def chunk(i, _):
            finish(issue(i, 0), 0)
            return 0
        lax.fori_loop(0, nch, chunk, 0)
    else:
        def chunk_pair(i, _):
            d0 = issue(2 * i, 0)
            d1 = issue(2 * i + 1, 1)
            finish(d0, 0)
            finish(d1, 1)
            return 0
        lax.fori_loop(0, nch // 2, chunk_pair, 0)
        if nch % 2:
            finish(issue(nch - 1, 0), 0)

    plsc.subcore_barrier()
    pltpu.sync_copy(agg_sp.at[pl.ds(s * ROWS, ROWS)], agg_stage)
    pltpu.sync_copy(agg_stage, agg_out.at[c, pl.ds(s * ROWS, ROWS)])
    pltpu.sync_copy(den_sp.at[pl.ds(s * ROWS, ROWS)], den_stage)
    pltpu.sync_copy(den_stage, den_out.at[c, pl.ds(s * ROWS, ROWS)])


@functools.cache
def _sc_fused(dp, nbuf):
    mesh = plsc.VectorSubcoreMesh(core_axis_name="c", subcore_axis_name="s")
    scratch = [
        pltpu.VMEM((L,), jnp.float32),
        pltpu.VMEM((ROWS, dp), jnp.float32),
        pltpu.VMEM((ROWS, L), jnp.float32),
        pltpu.VMEM_SHARED((NP, dp), jnp.float32),
        pltpu.VMEM_SHARED((NP, L), jnp.float32),
    ]
    for _ in range(nbuf):
        scratch += [
            pltpu.VMEM((C,), jnp.int32), pltpu.VMEM((C,), jnp.int32),
            pltpu.VMEM((C, dp), jnp.float32), pltpu.VMEM((C, dp), jnp.float32),
            pltpu.VMEM((C, dp), jnp.float32), pltpu.VMEM((C, L), jnp.float32),
            pltpu.SemaphoreType.DMA,
        ]
    return pl.kernel(
        functools.partial(_sc_fused_body, dp, nbuf),
        out_type=(jax.ShapeDtypeStruct((NC, NP, dp), jnp.float32),
                  jax.ShapeDtypeStruct((NC, NP, L), jnp.float32)),
        mesh=mesh,
        scratch_types=scratch,
        compiler_params=pltpu.CompilerParams(use_tc_tiling_on_sc=False),
    )


def _sc_wden_body(dp, d_true, q_hbm, k_hbm, src_hbm, dst_hbm,
                  w_out, den_out, src_i, dst_i, q_rows, k_rows,
                  w_buf, den_stage, den_sp, sem):
    c = lax.axis_index("c")
    s = lax.axis_index("s")
    base = (c * NS + s) * EP
    inv = 1.0 / math.sqrt(float(d_true))

    _zero_rows(den_stage, ROWS, L)
    pltpu.sync_copy(den_stage, den_sp.at[pl.ds(s * ROWS, ROWS)])
    plsc.subcore_barrier()

    def chunk(i, _):
        off = base + i * C
        pltpu.sync_copy(src_hbm.at[pl.ds(off, C)], src_i)
        pltpu.sync_copy(dst_hbm.at[pl.ds(off, C)], dst_i)
        dq = pltpu.async_copy(q_hbm.at[dst_i], q_rows, sem)
        dk = pltpu.async_copy(k_hbm.at[src_i], k_rows, sem)
        dq.wait()
        dk.wait()

        def edge(e, _):
            acc = q_rows[e, pl.ds(0, L)] * k_rows[e, pl.ds(0, L)]
            for t in range(1, dp // L):
                acc = acc + (q_rows[e, pl.ds(t * L, L)] *
                             k_rows[e, pl.ds(t * L, L)])
            w_buf[e, :] = jnp.exp(_lane_sum(acc) * inv)
            return 0
        lax.fori_loop(0, C, edge, 0)

        pltpu.sync_copy(w_buf, w_out.at[pl.ds(off, C)])
        pltpu.sync_copy(w_buf, den_sp.at[dst_i], add=True)
        return 0
    lax.fori_loop(0, EP // C, chunk, 0)

    plsc.subcore_barrier()
    pltpu.sync_copy(den_sp.at[pl.ds(s * ROWS, ROWS)], den_stage)
    pltpu.sync_copy(den_stage, den_out.at[c, pl.ds(s * ROWS, ROWS)])


@functools.cache
def _sc_wden(dp, d_true):
    mesh = plsc.VectorSubcoreMesh(core_axis_name="c", subcore_axis_name="s")
    return pl.kernel(
        functools.partial(_sc_wden_body, dp, d_true),
        out_type=(jax.ShapeDtypeStruct((E, L), jnp.float32),
                  jax.ShapeDtypeStruct((NC, NP, L), jnp.float32)),
        mesh=mesh,
        scratch_types=[
            pltpu.VMEM((C,), jnp.int32), pltpu.VMEM((C,), jnp.int32),
            pltpu.VMEM((C, dp), jnp.float32), pltpu.VMEM((C, dp), jnp.float32),
            pltpu.VMEM((C, L), jnp.float32),
            pltpu.VMEM((ROWS, L), jnp.float32),
            pltpu.VMEM_SHARED((NP, L), jnp.float32),
            pltpu.SemaphoreType.DMA,
        ],
        compiler_params=pltpu.CompilerParams(use_tc_tiling_on_sc=False),
    )


def _sc_agg_body(dpc, ncc, v_hbm, src_hbm, dst_hbm, w_hbm,
                 agg_out, src_i, dst_i, idx2, v_rows, w_buf, agg_stage,
                 agg_sp, sem):
    c = lax.axis_index("c")
    s = lax.axis_index("s")
    base = (c * NS + s) * EP

    _zero_rows(agg_stage, ROWS, dpc)
    for j in range(ncc):
        pltpu.sync_copy(agg_stage, agg_sp.at[pl.ds(s * ROWS, ROWS)])
        plsc.subcore_barrier()

        def chunk(i, _):
            off = base + i * C
            pltpu.sync_copy(src_hbm.at[pl.ds(off, C)], src_i)
            pltpu.sync_copy(dst_hbm.at[pl.ds(off, C)], dst_i)
            pltpu.sync_copy(w_hbm.at[pl.ds(off, C)], w_buf)
            for t in range(C // L):
                idx2[pl.ds(t * L, L)] = src_i[pl.ds(t * L, L)] * ncc + j
            pltpu.async_copy(v_hbm.at[idx2], v_rows, sem).wait()

            def edge(e, _):
                w = w_buf[e, :]
                for t in range(dpc // L):
                    v_rows[e, pl.ds(t * L, L)] = (
                        v_rows[e, pl.ds(t * L, L)] * w)
                return 0
            lax.fori_loop(0, C, edge, 0)

            pltpu.sync_copy(v_rows, agg_sp.at[dst_i], add=True)
            return 0
        lax.fori_loop(0, EP // C, chunk, 0)

        plsc.subcore_barrier()
        pltpu.sync_copy(agg_sp.at[pl.ds(s * ROWS, ROWS)], agg_stage)
        pltpu.sync_copy(
            agg_stage, agg_out.at[c, pl.ds(j * NP + s * ROWS, ROWS)])
        if j != ncc - 1:
            _zero_rows(agg_stage, ROWS, dpc)


@functools.cache
def _sc_agg(dpc, ncc):
    mesh = plsc.VectorSubcoreMesh(core_axis_name="c", subcore_axis_name="s")
    return pl.kernel(
        functools.partial(_sc_agg_body, dpc, ncc),
        out_type=jax.ShapeDtypeStruct((NC, ncc * NP, dpc), jnp.float32),
        mesh=mesh,
        scratch_types=[
            pltpu.VMEM((C,), jnp.int32), pltpu.VMEM((C,), jnp.int32),
            pltpu.VMEM((C,), jnp.int32),
            pltpu.VMEM((C, dpc), jnp.float32),
            pltpu.VMEM((C, L), jnp.float32),
            pltpu.VMEM((ROWS, dpc), jnp.float32),
            pltpu.VMEM_SHARED((NP, dpc), jnp.float32),
            pltpu.SemaphoreType.DMA,
        ],
        compiler_params=pltpu.CompilerParams(use_tc_tiling_on_sc=False),
    )


def _combine_kernel(act, a_ref, d_ref, s_ref, o_ref):
    den = d_ref[0, :, 0:1] + d_ref[1, :, 0:1] + 1e-16
    val = (a_ref[0, 0] + a_ref[1, 0]) / den + s_ref[0]
    if act == 'relu':
        val = jnp.maximum(val, 0.0)
    elif act == 'elu':
        val = jnp.where(val > 0, val, jnp.exp(val) - 1.0)
    o_ref[0] = val


def _combine(agg, den, s_arr, act, ncc, dpc):
    dp = ncc * dpc
    agg = agg.reshape(NC, ncc, NP, dpc)
    s_t = s_arr.reshape(NP, ncc, dpc).transpose(1, 0, 2)
    out = pl.pallas_call(
        functools.partial(_combine_kernel, act),
        grid=(ncc, NP // _BN),
        in_specs=[
            pl.BlockSpec((NC, 1, _BN, dpc), lambda j, i: (0, j, i, 0)),
            pl.BlockSpec((NC, _BN, L), lambda j, i: (0, i, 0)),
            pl.BlockSpec((1, _BN, dpc), lambda j, i: (j, i, 0)),
        ],
        out_specs=pl.BlockSpec((1, _BN, dpc), lambda j, i: (j, i, 0)),
        out_shape=jax.ShapeDtypeStruct((ncc, NP, dpc), jnp.float32),
    )(agg, den, s_t)
    return out.transpose(1, 0, 2).reshape(NP, dp)


def _smscale_kernel(p_ref, e_ref, o0_ref, o1_ref):
    x0 = p_ref[:, 0:1]
    x1 = p_ref[:, 1:2]
    m = jnp.maximum(x0, x1)
    e0 = jnp.exp(x0 - m)
    e1 = jnp.exp(x1 - m)
    p0 = e0 / (e0 + e1)
    emb = e_ref[...]
    o0_ref[...] = p0 * emb
    o1_ref[...] = (1.0 - p0) * emb


def _smscale(pred, emb):
    nn, dd = emb.shape
    dw = pred.shape[1]
    return pl.pallas_call(
        _smscale_kernel,
        grid=(nn // _BN,),
        in_specs=[pl.BlockSpec((_BN, dw), lambda i: (i, 0)),
                  pl.BlockSpec((_BN, dd), lambda i: (i, 0))],
        out_specs=[pl.BlockSpec((_BN, dd), lambda i: (i, 0)),
                   pl.BlockSpec((_BN, dd), lambda i: (i, 0))],
        out_shape=[jax.ShapeDtypeStruct((nn, dd), jnp.float32),
                   jax.ShapeDtypeStruct((nn, dd), jnp.float32)],
    )(pred, emb)


def _tconv(x, esrc, edst, p, act):
    d = p['Wq'].shape[0]
    dp = 32
    q, k, v, s = _proj(x, p, dp)
    inv = jnp.full((L,), 1.0 / math.sqrt(float(d)), jnp.float32)
    agg, den = _sc_fused(dp, 2)(q, k, v, inv, esrc, edst)
    return _combine(agg, den, s, act, 1, dp)


def _tconv64(x, esrc, edst, p, act):
    d = p['Wq'].shape[0]
    q, k, v, s = _proj(x, p, d)
    inv = jnp.full((L,), 1.0 / math.sqrt(float(d)), jnp.float32)
    agg, den = _sc_fused(64, 1)(q, k, v, inv, esrc, edst)
    return _combine(agg, den, s, act, 1, d)


def _tconv_wide(x, esrc, edst, p, act):
    d = p['Wq'].shape[0]
    dpc = 32
    ncc = d // dpc
    q, k, v, s = _proj(x, p, d)
    w_e, den = _sc_wden(d, d)(q, k, esrc, edst)
    v_view = v.reshape(NP * ncc, dpc)
    agg = _sc_agg(dpc, ncc)(v_view, esrc, edst, w_e)
    return _combine(agg, den, s, act, ncc, dpc)


def _head(x, esrc, edst, plist):
    h = x
    for p in plist:
        h = _tconv(h, esrc, edst, p, 'relu')
    return h


def _graph_forward(x, edge, params):
    esrc = edge[0].astype(jnp.int32)
    edst = edge[1].astype(jnp.int32)
    h = _tconv_wide(x, esrc, edst, params['fe'][0], 'elu')
    emb = _tconv64(h, esrc, edst, params['fe'][1], 'elu')
    pred = _head(emb, esrc, edst, params['cls'])
    dom = _head(emb, esrc, edst, params['gdc'])
    fs0, fs1 = _smscale(pred, emb)
    d0 = _head(fs0, esrc, edst, params['dcis'][0])
    d1 = _head(fs1, esrc, edst, params['dcis'][1])
    return pred, dom, d0, d1, emb


def kernel(src_x, tar_x, params, src_edge, tar_edge):
    xs = jnp.pad(src_x, ((0, NP - N), (0, 0)))
    xt = jnp.pad(tar_x, ((0, NP - N), (0, 0)))

    s_pred, s_dom, s_d0, s_d1, s_emb = _graph_forward(xs, src_edge, params)
    t_pred, t_dom, t_d0, t_d1, t_emb = _graph_forward(xt, tar_edge, params)

    return (s_pred[:N, :2], s_dom[:N, :2], t_dom[:N, :2],
            s_d0[:N, :2], s_d1[:N, :2], t_d0[:N, :2], t_d1[:N, :2],
            s_emb[:N], t_emb[:N])
